# baseline (speedup 1.0000x reference)
import functools

import jax
import jax.numpy as jnp
from jax import lax
from jax.experimental import pallas as pl
from jax.experimental.pallas import tpu as pltpu
from jax.experimental.pallas import tpu_sc as plsc

D = 128


_INFO = plsc.get_sparse_core_info()
_NC, _NS = _INFO.num_cores, _INFO.num_subcores
_NW = _NC * _NS
_NBUF = 4


@functools.lru_cache(maxsize=None)
def _make_gather(V, counts):
    max_rows = max(counts)
    mesh = plsc.VectorSubcoreMesh(core_axis_name="c", subcore_axis_name="s")

    out_type = tuple(
        jax.ShapeDtypeStruct((c * _NW * 128, D), jnp.float32) for c in counts
    )

    @functools.partial(
        pl.kernel,
        out_type=out_type,
        mesh=mesh,
        scratch_types=[pltpu.VMEM((max_rows * 128,), jnp.int32)]
        + [pltpu.VMEM((128, D), jnp.float32) for _ in range(_NBUF)]
        + [pltpu.SemaphoreType.DMA for _ in range(2 * _NBUF)],
    )
    def gather(table_hbm, i0, i1, i2, i3, o0, o1, o2, o3, idx_v, *bufs_sems):
        bufs = bufs_sems[:_NBUF]
        gsems = bufs_sems[_NBUF:2 * _NBUF]
        ssems = bufs_sems[2 * _NBUF:]
        wid = lax.axis_index("s") * _NC + lax.axis_index("c")
        idx_refs = (i0, i1, i2, i3)
        outs = (o0, o1, o2, o3)

        def pipeline(out, nr, base_r):
            G = _NBUF // 2

            def issue(j, b):
                pltpu.async_copy(
                    table_hbm.at[idx_v.at[pl.ds(j * 128, 128)]], bufs[b],
                    gsems[b])

            def wait_g(b):
                pltpu.make_async_copy(
                    table_hbm.at[pl.ds(0, 128)], bufs[b], gsems[b]).wait()

            def store(j, b):
                pltpu.async_copy(
                    bufs[b], out.at[pl.ds((base_r + j) * 128, 128)],
                    ssems[b])

            def wait_s(b):
                pltpu.make_async_copy(
                    bufs[b], out.at[pl.ds(0, 128)], ssems[b]).wait()

            for b in range(G):
                issue(b, b)
            for j in range(G):
                issue(j + G, (j + G) % _NBUF)
                wait_g(j % _NBUF)
                store(j, j % _NBUF)

            steady_n = nr - 2 * G
            nk = (steady_n + _NBUF - 1) // _NBUF

            def body(k, _):
                for u in range(_NBUF):
                    j = G + k * _NBUF + u

                    @pl.when(j < nr - G)
                    def _(j=j, u=u):
                        b = (G + u) % _NBUF
                        br = (2 * G + u) % _NBUF
                        wait_s(br)
                        issue(j + G, br)
                        wait_g(b)
                        store(j, b)
                return 0

            lax.fori_loop(0, nk, body, 0)
            for j in range(nr - G, nr):
                wait_g(j % _NBUF)
                store(j, j % _NBUF)
            for j in range(nr - _NBUF, nr):
                wait_s(j % _NBUF)

        for seg in range(4):
            nr = counts[seg]
            base_r = wid * nr
            pltpu.sync_copy(idx_refs[seg].at[pl.ds(base_r * 128, nr * 128)],
                            idx_v.at[pl.ds(0, nr * 128)])
            pipeline(outs[seg], nr, base_r)

    return gather


def _finish_body(bn, ns, ns_p, rows_ref, w_ref, b_ref,
                 ex_ref, px_ref, ey_ref, py_ref):
    e = rows_ref[...]
    e2 = e.reshape(bn * ns_p, D)
    p2 = lax.dot_general(
        jnp.maximum(e2, 0.0), w_ref[...],
        dimension_numbers=(((1,), (1,)), ((), ())),
        preferred_element_type=jnp.float32,
    ) + b_ref[...]
    p = p2.reshape(bn, ns_p, D)
    e3 = e[:, :ns, :]
    p3 = p[:, :ns, :]
    ex_ref[...] = e3
    ey_ref[...] = e3
    px_ref[...] = p3
    py_ref[...] = p3


def _finish(rows3, W, b, B, ns, ns_p):
    bn = 64
    shp = jax.ShapeDtypeStruct((B, ns, D), jnp.float32)
    o3 = pl.BlockSpec((bn, ns, D), lambda i: (i, 0, 0))
    return pl.pallas_call(
        functools.partial(_finish_body, bn, ns, ns_p),
        grid=(B // bn,),
        in_specs=[
            pl.BlockSpec((bn, ns_p, D), lambda i: (i, 0, 0)),
            pl.BlockSpec((D, D), lambda i: (0, 0)),
            pl.BlockSpec((1, D), lambda i: (0, 0)),
        ],
        out_specs=[o3, o3, o3, o3],
        out_shape=[shp, shp, shp, shp],
    )(rows3, W, b.reshape(1, D))


def _pad_cols(n):
    return -(-n // 8) * 8


def kernel(tensor_demo, tensor_med, tensor_vitals, tensor_labs, table, W, b):
    V = table.shape[0]
    tensors = (tensor_demo, tensor_med, tensor_vitals, tensor_labs)
    idxs = []
    counts = []
    for t in tensors:
        B, ns = t.shape
        ns_p = _pad_cols(ns)
        tp = jnp.pad(t.astype(jnp.int32), ((0, 0), (0, ns_p - ns)))
        counts.append(B * ns_p // (128 * _NW))
        idxs.append(tp.reshape(B * ns_p))

    rows = _make_gather(V, tuple(counts))(table, *idxs)

    embs_x, projs_x, embs_y, projs_y = [], [], [], []
    for r, t in zip(rows, tensors):
        B, ns = t.shape
        ns_p = _pad_cols(ns)
        ex, px, ey, py = _finish(r.reshape(B, ns_p, D), W, b, B, ns, ns_p)
        embs_x.append(ex)
        projs_x.append(px)
        embs_y.append(ey)
        projs_y.append(py)
    return (tuple(embs_x), tuple(projs_x), tuple(embs_y), tuple(projs_y))


# speedup vs baseline: 2.9289x; 2.9289x over previous
import functools

import jax
import jax.numpy as jnp
from jax import lax
from jax.experimental import pallas as pl
from jax.experimental.pallas import tpu as pltpu
from jax.experimental.pallas import tpu_sc as plsc

D = 128


_INFO = plsc.get_sparse_core_info()
_NC, _NS = _INFO.num_cores, _INFO.num_subcores
_NW = _NC * _NS
_NBUF = 4


@functools.lru_cache(maxsize=None)
def _make_gather(V, counts):
    max_rows = max(counts)
    mesh = plsc.VectorSubcoreMesh(core_axis_name="c", subcore_axis_name="s")

    out_type = tuple(
        jax.ShapeDtypeStruct((c * _NW * 128, D), jnp.float32) for c in counts
    )

    @functools.partial(
        pl.kernel,
        out_type=out_type,
        mesh=mesh,
        scratch_types=[pltpu.VMEM((max_rows * 128,), jnp.int32)]
        + [pltpu.VMEM((128, D), jnp.float32) for _ in range(_NBUF)]
        + [pltpu.SemaphoreType.DMA for _ in range(2 * _NBUF)],
    )
    def gather(table_hbm, i0, i1, i2, i3, o0, o1, o2, o3, idx_v, *bufs_sems):
        bufs = bufs_sems[:_NBUF]
        gsems = bufs_sems[_NBUF:2 * _NBUF]
        ssems = bufs_sems[2 * _NBUF:]
        wid = lax.axis_index("s") * _NC + lax.axis_index("c")
        idx_refs = (i0, i1, i2, i3)
        outs = (o0, o1, o2, o3)

        def pipeline(out, nr, base_r):
            G = _NBUF // 2

            def issue(j, b):
                pltpu.async_copy(
                    table_hbm.at[idx_v.at[pl.ds(j * 128, 128)]], bufs[b],
                    gsems[b])

            def wait_g(b):
                pltpu.make_async_copy(
                    table_hbm.at[pl.ds(0, 128)], bufs[b], gsems[b]).wait()

            def store(j, b):
                pltpu.async_copy(
                    bufs[b], out.at[pl.ds((base_r + j) * 128, 128)],
                    ssems[b])

            def wait_s(b):
                pltpu.make_async_copy(
                    bufs[b], out.at[pl.ds(0, 128)], ssems[b]).wait()

            for b in range(G):
                issue(b, b)
            for j in range(G):
                issue(j + G, (j + G) % _NBUF)
                wait_g(j % _NBUF)
                store(j, j % _NBUF)

            steady_n = nr - 2 * G
            nk = (steady_n + _NBUF - 1) // _NBUF

            def body(k, _):
                for u in range(_NBUF):
                    j = G + k * _NBUF + u

                    @pl.when(j < nr - G)
                    def _(j=j, u=u):
                        b = (G + u) % _NBUF
                        br = (2 * G + u) % _NBUF
                        wait_s(br)
                        issue(j + G, br)
                        wait_g(b)
                        store(j, b)
                return 0

            lax.fori_loop(0, nk, body, 0)
            for j in range(nr - G, nr):
                wait_g(j % _NBUF)
                store(j, j % _NBUF)
            for j in range(nr - _NBUF, nr):
                wait_s(j % _NBUF)

        for seg in range(4):
            nr = counts[seg]
            base_r = wid * nr
            pltpu.sync_copy(idx_refs[seg].at[pl.ds(base_r * 128, nr * 128)],
                            idx_v.at[pl.ds(0, nr * 128)])
            pipeline(outs[seg], nr, base_r)

    return gather


def _finish_body(bn, ns, ns_p, rows_ref, w_ref, b_ref,
                 ex_ref, px_ref, ey_ref, py_ref):
    e = rows_ref[...]
    e2 = e.reshape(bn * ns_p, D)
    p2 = lax.dot_general(
        jnp.maximum(e2, 0.0), w_ref[...],
        dimension_numbers=(((1,), (1,)), ((), ())),
        preferred_element_type=jnp.float32,
    ) + b_ref[...]
    p = p2.reshape(bn, ns_p, D)
    e3 = e[:, :ns, :]
    p3 = p[:, :ns, :]
    ex_ref[...] = e3
    ey_ref[...] = e3
    px_ref[...] = p3
    py_ref[...] = p3


def _finish(rows3, W, b, B, ns, ns_p):
    bn = 64
    shp = jax.ShapeDtypeStruct((B, ns, D), jnp.float32)
    o3 = pl.BlockSpec((bn, ns, D), lambda i: (i, 0, 0))
    return pl.pallas_call(
        functools.partial(_finish_body, bn, ns, ns_p),
        grid=(B // bn,),
        in_specs=[
            pl.BlockSpec((bn, ns_p, D), lambda i: (i, 0, 0)),
            pl.BlockSpec((D, D), lambda i: (0, 0)),
            pl.BlockSpec((1, D), lambda i: (0, 0)),
        ],
        out_specs=[o3, o3, o3, o3],
        out_shape=[shp, shp, shp, shp],
    )(rows3, W, b.reshape(1, D))


def _pad_cols(n):
    return -(-n // 8) * 8


def kernel(tensor_demo, tensor_med, tensor_vitals, tensor_labs, table, W, b):
    V = table.shape[0]
    tensors = (tensor_demo, tensor_med, tensor_vitals, tensor_labs)
    idxs = []
    counts = []
    for t in tensors:
        B, ns = t.shape
        ns_p = _pad_cols(ns)
        t32 = t.astype(jnp.int32)
        tp = jnp.concatenate([t32, t32[:, :ns_p - ns]], axis=1)
        counts.append(B * ns_p // (128 * _NW))
        idxs.append(tp.reshape(B * ns_p))

    rows = _make_gather(V, tuple(counts))(table, *idxs)

    embs_x, projs_x, embs_y, projs_y = [], [], [], []
    for r, t in zip(rows, tensors):
        B, ns = t.shape
        ns_p = _pad_cols(ns)
        ex, px, ey, py = _finish(r.reshape(B, ns_p, D), W, b, B, ns, ns_p)
        embs_x.append(ex)
        projs_x.append(px)
        embs_y.append(ey)
        projs_y.append(py)
    return (tuple(embs_x), tuple(projs_x), tuple(embs_y), tuple(projs_y))


# speedup vs baseline: 5.2281x; 1.7850x over previous
import functools

import jax
import jax.numpy as jnp
from jax import lax
from jax.experimental import pallas as pl
from jax.experimental.pallas import tpu as pltpu
from jax.experimental.pallas import tpu_sc as plsc

D = 128


_INFO = plsc.get_sparse_core_info()
_NC, _NS = _INFO.num_cores, _INFO.num_subcores
_NW = _NC * _NS
_NBUF = 4


@functools.lru_cache(maxsize=None)
def _make_gather(V, counts):
    max_rows = max(counts)
    mesh = plsc.VectorSubcoreMesh(core_axis_name="c", subcore_axis_name="s")

    out_type = tuple(
        jax.ShapeDtypeStruct((c * _NW * 128, D), jnp.float32) for c in counts
    )

    @functools.partial(
        pl.kernel,
        out_type=out_type,
        mesh=mesh,
        scratch_types=[pltpu.VMEM((max_rows * 128,), jnp.int32)]
        + [pltpu.VMEM((128, D), jnp.float32) for _ in range(_NBUF)]
        + [pltpu.SemaphoreType.DMA for _ in range(2 * _NBUF)],
    )
    def gather(table_hbm, i0, i1, i2, i3, o0, o1, o2, o3, idx_v, *bufs_sems):
        bufs = bufs_sems[:_NBUF]
        gsems = bufs_sems[_NBUF:2 * _NBUF]
        ssems = bufs_sems[2 * _NBUF:]
        wid = lax.axis_index("s") * _NC + lax.axis_index("c")
        idx_refs = (i0, i1, i2, i3)
        outs = (o0, o1, o2, o3)

        def pipeline(out, nr, base_r):
            G = _NBUF // 2

            def issue(j, b):
                pltpu.async_copy(
                    table_hbm.at[idx_v.at[pl.ds(j * 128, 128)]], bufs[b],
                    gsems[b])

            def wait_g(b):
                pltpu.make_async_copy(
                    table_hbm.at[pl.ds(0, 128)], bufs[b], gsems[b]).wait()

            def store(j, b):
                pltpu.async_copy(
                    bufs[b], out.at[pl.ds((base_r + j) * 128, 128)],
                    ssems[b])

            def wait_s(b):
                pltpu.make_async_copy(
                    bufs[b], out.at[pl.ds(0, 128)], ssems[b]).wait()

            for b in range(G):
                issue(b, b)
            for j in range(G):
                issue(j + G, (j + G) % _NBUF)
                wait_g(j % _NBUF)
                store(j, j % _NBUF)

            steady_n = nr - 2 * G
            nk = (steady_n + _NBUF - 1) // _NBUF

            def body(k, _):
                for u in range(_NBUF):
                    j = G + k * _NBUF + u

                    @pl.when(j < nr - G)
                    def _(j=j, u=u):
                        b = (G + u) % _NBUF
                        br = (2 * G + u) % _NBUF
                        wait_s(br)
                        issue(j + G, br)
                        wait_g(b)
                        store(j, b)
                return 0

            lax.fori_loop(0, nk, body, 0)
            for j in range(nr - G, nr):
                wait_g(j % _NBUF)
                store(j, j % _NBUF)
            for j in range(nr - _NBUF, nr):
                wait_s(j % _NBUF)

        for seg in range(4):
            nr = counts[seg]
            base_r = wid * nr
            pltpu.sync_copy(idx_refs[seg].at[pl.ds(base_r * 128, nr * 128)],
                            idx_v.at[pl.ds(0, nr * 128)])
            pipeline(outs[seg], nr, base_r)

    return gather


def _finish_body(rows_ref, w_ref, b_ref, ex_ref, px_ref, ey_ref, py_ref):
    e = rows_ref[...]
    p = lax.dot_general(
        jnp.maximum(e, 0.0), w_ref[...],
        dimension_numbers=(((1,), (1,)), ((), ())),
        preferred_element_type=jnp.float32,
    ) + b_ref[...]
    ex_ref[...] = e
    ey_ref[...] = e
    px_ref[...] = p
    py_ref[...] = p


def _finish(rows, W, b):
    N = rows.shape[0]
    R = 1024
    shp = jax.ShapeDtypeStruct((N, D), jnp.float32)
    o2 = pl.BlockSpec((R, D), lambda i: (i, 0))
    return pl.pallas_call(
        _finish_body,
        grid=(N // R,),
        in_specs=[
            pl.BlockSpec((R, D), lambda i: (i, 0)),
            pl.BlockSpec((D, D), lambda i: (0, 0)),
            pl.BlockSpec((1, D), lambda i: (0, 0)),
        ],
        out_specs=[o2, o2, o2, o2],
        out_shape=[shp, shp, shp, shp],
    )(rows, W, b.reshape(1, D))


def kernel(tensor_demo, tensor_med, tensor_vitals, tensor_labs, table, W, b):
    V = table.shape[0]
    tensors = (tensor_demo, tensor_med, tensor_vitals, tensor_labs)
    idxs = []
    counts = []
    for t in tensors:
        B, ns = t.shape
        counts.append(B * ns // (128 * _NW))
        idxs.append(t.astype(jnp.int32).T.reshape(B * ns))

    rows = _make_gather(V, tuple(counts))(table, *idxs)

    embs_x, projs_x, embs_y, projs_y = [], [], [], []
    for r, t in zip(rows, tensors):
        B, ns = t.shape
        ex, px, ey, py = _finish(r, W, b)
        for acc, a in ((embs_x, ex), (projs_x, px), (embs_y, ey),
                       (projs_y, py)):
            acc.append(jnp.transpose(a.reshape(ns, B, D), (1, 0, 2)))
    return (tuple(embs_x), tuple(projs_x), tuple(embs_y), tuple(projs_y))


# speedup vs baseline: 5.4378x; 1.0401x over previous
import functools

import jax
import jax.numpy as jnp
from jax import lax
from jax.experimental import pallas as pl
from jax.experimental.pallas import tpu as pltpu
from jax.experimental.pallas import tpu_sc as plsc

D = 128


_INFO = plsc.get_sparse_core_info()
_NC, _NS = _INFO.num_cores, _INFO.num_subcores
_NW = _NC * _NS
_NBUF = 4


@functools.lru_cache(maxsize=None)
def _make_gather(V, nr):
    mesh = plsc.VectorSubcoreMesh(core_axis_name="c", subcore_axis_name="s")

    @functools.partial(
        pl.kernel,
        out_type=jax.ShapeDtypeStruct((nr * _NW * 128, D), jnp.float32),
        mesh=mesh,
        scratch_types=[pltpu.VMEM((nr * 128,), jnp.int32)]
        + [pltpu.VMEM((128, D), jnp.float32) for _ in range(_NBUF)]
        + [pltpu.SemaphoreType.DMA for _ in range(2 * _NBUF)],
    )
    def gather(table_hbm, idx_hbm, out, idx_v, *bufs_sems):
        bufs = bufs_sems[:_NBUF]
        gsems = bufs_sems[_NBUF:2 * _NBUF]
        ssems = bufs_sems[2 * _NBUF:]
        wid = lax.axis_index("s") * _NC + lax.axis_index("c")

        def pipeline(out, nr, base_r):
            G = _NBUF // 2

            def issue(j, b):
                pltpu.async_copy(
                    table_hbm.at[idx_v.at[pl.ds(j * 128, 128)]], bufs[b],
                    gsems[b])

            def wait_g(b):
                pltpu.make_async_copy(
                    table_hbm.at[pl.ds(0, 128)], bufs[b], gsems[b]).wait()

            def store(j, b):
                pltpu.async_copy(
                    bufs[b], out.at[pl.ds((base_r + j) * 128, 128)],
                    ssems[b])

            def wait_s(b):
                pltpu.make_async_copy(
                    bufs[b], out.at[pl.ds(0, 128)], ssems[b]).wait()

            for b in range(G):
                issue(b, b)
            for j in range(G):
                issue(j + G, (j + G) % _NBUF)
                wait_g(j % _NBUF)
                store(j, j % _NBUF)

            steady_n = nr - 2 * G
            nk = (steady_n + _NBUF - 1) // _NBUF

            def body(k, _):
                for u in range(_NBUF):
                    j = G + k * _NBUF + u

                    @pl.when(j < nr - G)
                    def _(j=j, u=u):
                        b = (G + u) % _NBUF
                        br = (2 * G + u) % _NBUF
                        wait_s(br)
                        issue(j + G, br)
                        wait_g(b)
                        store(j, b)
                return 0

            lax.fori_loop(0, nk, body, 0)
            for j in range(nr - G, nr):
                wait_g(j % _NBUF)
                store(j, j % _NBUF)
            for j in range(nr - _NBUF, nr):
                wait_s(j % _NBUF)

        base_r = wid * nr
        pltpu.sync_copy(idx_hbm.at[pl.ds(base_r * 128, nr * 128)],
                        idx_v.at[pl.ds(0, nr * 128)])
        pipeline(out, nr, base_r)

    return gather


def _finish_body(rows_ref, w_ref, b_ref, ex_ref, px_ref, ey_ref, py_ref):
    e = rows_ref[...]
    p = lax.dot_general(
        jnp.maximum(e, 0.0), w_ref[...],
        dimension_numbers=(((1,), (1,)), ((), ())),
        preferred_element_type=jnp.float32,
    ) + b_ref[...]
    ex_ref[...] = e
    ey_ref[...] = e
    px_ref[...] = p
    py_ref[...] = p


def _finish(rows, W, b):
    N = rows.shape[0]
    R = 1024
    shp = jax.ShapeDtypeStruct((N, D), jnp.float32)
    o2 = pl.BlockSpec((R, D), lambda i: (i, 0))
    return pl.pallas_call(
        _finish_body,
        grid=(N // R,),
        in_specs=[
            pl.BlockSpec((R, D), lambda i: (i, 0)),
            pl.BlockSpec((D, D), lambda i: (0, 0)),
            pl.BlockSpec((1, D), lambda i: (0, 0)),
        ],
        out_specs=[o2, o2, o2, o2],
        out_shape=[shp, shp, shp, shp],
    )(rows, W, b.reshape(1, D))


def kernel(tensor_demo, tensor_med, tensor_vitals, tensor_labs, table, W, b):
    V = table.shape[0]
    tensors = (tensor_demo, tensor_med, tensor_vitals, tensor_labs)
    embs_x, projs_x, embs_y, projs_y = [], [], [], []
    for t in tensors:
        B, ns = t.shape
        nr = B * ns // (128 * _NW)
        idx = t.astype(jnp.int32).T.reshape(B * ns)
        r = _make_gather(V, nr)(table, idx)
        ex, px, ey, py = _finish(r, W, b)
        for acc, a in ((embs_x, ex), (projs_x, px), (embs_y, ey),
                       (projs_y, py)):
            acc.append(jnp.transpose(a.reshape(ns, B, D), (1, 0, 2)))
    return (tuple(embs_x), tuple(projs_x), tuple(embs_y), tuple(projs_y))


# speedup vs baseline: 6.1061x; 1.1229x over previous
import functools

import jax
import jax.numpy as jnp
from jax import lax
from jax.experimental import pallas as pl
from jax.experimental.pallas import tpu as pltpu
from jax.experimental.pallas import tpu_sc as plsc

D = 128


_INFO = plsc.get_sparse_core_info()
_NC, _NS = _INFO.num_cores, _INFO.num_subcores
_NW = _NC * _NS
_NBUF = 4


@functools.lru_cache(maxsize=None)
def _make_gather(V, nr):
    mesh = plsc.VectorSubcoreMesh(core_axis_name="c", subcore_axis_name="s")

    rows_t = jax.ShapeDtypeStruct((nr * _NW * 128, D), jnp.float32)

    @functools.partial(
        pl.kernel,
        out_type=(rows_t, rows_t),
        mesh=mesh,
        scratch_types=[pltpu.VMEM((nr * 128,), jnp.int32)]
        + [pltpu.VMEM((128, D), jnp.float32) for _ in range(_NBUF)]
        + [pltpu.SemaphoreType.DMA for _ in range(2 * _NBUF)],
    )
    def gather(table_hbm, idx_hbm, out_x, out_y, idx_v, *bufs_sems):
        bufs = bufs_sems[:_NBUF]
        gsems = bufs_sems[_NBUF:2 * _NBUF]
        ssems = bufs_sems[2 * _NBUF:]
        wid = lax.axis_index("s") * _NC + lax.axis_index("c")
        outs = (out_x, out_y)

        def pipeline(nr, base_r):
            G = _NBUF // 2

            def issue(j, b):
                pltpu.async_copy(
                    table_hbm.at[idx_v.at[pl.ds(j * 128, 128)]], bufs[b],
                    gsems[b])

            def wait_g(b):
                pltpu.make_async_copy(
                    table_hbm.at[pl.ds(0, 128)], bufs[b], gsems[b]).wait()

            def store(j, b):
                for out in outs:
                    pltpu.async_copy(
                        bufs[b], out.at[pl.ds((base_r + j) * 128, 128)],
                        ssems[b])

            def wait_s(b):
                for out in outs:
                    pltpu.make_async_copy(
                        bufs[b], out.at[pl.ds(0, 128)], ssems[b]).wait()

            for b in range(G):
                issue(b, b)
            for j in range(G):
                issue(j + G, (j + G) % _NBUF)
                wait_g(j % _NBUF)
                store(j, j % _NBUF)

            steady_n = nr - 2 * G
            nk = (steady_n + _NBUF - 1) // _NBUF

            def body(k, _):
                for u in range(_NBUF):
                    j = G + k * _NBUF + u

                    @pl.when(j < nr - G)
                    def _(j=j, u=u):
                        b = (G + u) % _NBUF
                        br = (2 * G + u) % _NBUF
                        wait_s(br)
                        issue(j + G, br)
                        wait_g(b)
                        store(j, b)
                return 0

            lax.fori_loop(0, nk, body, 0)
            for j in range(nr - G, nr):
                wait_g(j % _NBUF)
                store(j, j % _NBUF)
            for j in range(nr - _NBUF, nr):
                wait_s(j % _NBUF)

        base_r = wid * nr
        pltpu.sync_copy(idx_hbm.at[pl.ds(base_r * 128, nr * 128)],
                        idx_v.at[pl.ds(0, nr * 128)])
        pipeline(nr, base_r)

    return gather


def _finish_body(rows_ref, w_ref, b_ref, px_ref, py_ref):
    e = rows_ref[...]
    p = lax.dot_general(
        jnp.maximum(e, 0.0), w_ref[...],
        dimension_numbers=(((1,), (1,)), ((), ())),
        preferred_element_type=jnp.float32,
    ) + b_ref[...]
    px_ref[...] = p
    py_ref[...] = p


def _finish(rows, W, b):
    N = rows.shape[0]
    R = 1024
    shp = jax.ShapeDtypeStruct((N, D), jnp.float32)
    o2 = pl.BlockSpec((R, D), lambda i: (i, 0))
    return pl.pallas_call(
        _finish_body,
        grid=(N // R,),
        in_specs=[
            pl.BlockSpec((R, D), lambda i: (i, 0)),
            pl.BlockSpec((D, D), lambda i: (0, 0)),
            pl.BlockSpec((1, D), lambda i: (0, 0)),
        ],
        out_specs=[o2, o2],
        out_shape=[shp, shp],
    )(rows, W, b.reshape(1, D))


def kernel(tensor_demo, tensor_med, tensor_vitals, tensor_labs, table, W, b):
    V = table.shape[0]
    tensors = (tensor_demo, tensor_med, tensor_vitals, tensor_labs)
    embs_x, projs_x, embs_y, projs_y = [], [], [], []
    for t in tensors:
        B, ns = t.shape
        nr = B * ns // (128 * _NW)
        idx = t.astype(jnp.int32).T.reshape(B * ns)
        ex, ey = _make_gather(V, nr)(table, idx)
        px, py = _finish(ex, W, b)
        for acc, a in ((embs_x, ex), (projs_x, px), (embs_y, ey),
                       (projs_y, py)):
            acc.append(jnp.transpose(a.reshape(ns, B, D), (1, 0, 2)))
    return (tuple(embs_x), tuple(projs_x), tuple(embs_y), tuple(projs_y))


# speedup vs baseline: 7.9479x; 1.3016x over previous
import functools

import jax
import jax.numpy as jnp
from jax import lax
from jax.experimental import pallas as pl
from jax.experimental.pallas import tpu as pltpu
from jax.experimental.pallas import tpu_sc as plsc

D = 128


_INFO = plsc.get_sparse_core_info()
_NC, _NS = _INFO.num_cores, _INFO.num_subcores
_NW = _NC * _NS


@functools.lru_cache(maxsize=None)
def _make_gather(V, nidx, C, nbuf):
    m = nidx // C
    mesh = plsc.VectorSubcoreMesh(core_axis_name="c", subcore_axis_name="s")

    rows_t = jax.ShapeDtypeStruct((nidx * _NW, D), jnp.float32)

    @functools.partial(
        pl.kernel,
        out_type=rows_t,
        mesh=mesh,
        scratch_types=[pltpu.VMEM((nidx,), jnp.int32)]
        + [pltpu.VMEM((C, D), jnp.float32) for _ in range(nbuf)]
        + [pltpu.SemaphoreType.DMA for _ in range(2 * nbuf)],
    )
    def gather(table_hbm, idx_hbm, out_x, idx_v, *bufs_sems):
        bufs = bufs_sems[:nbuf]
        gsems = bufs_sems[nbuf:2 * nbuf]
        ssems = bufs_sems[2 * nbuf:]
        wid = lax.axis_index("s") * _NC + lax.axis_index("c")
        outs = (out_x,)

        def pipeline(base):
            G = nbuf // 2

            def issue(j, b):
                pltpu.async_copy(
                    table_hbm.at[idx_v.at[pl.ds(j * C, C)]], bufs[b],
                    gsems[b])

            def wait_g(b):
                pltpu.make_async_copy(
                    table_hbm.at[pl.ds(0, C)], bufs[b], gsems[b]).wait()

            def store(j, b):
                for out in outs:
                    pltpu.async_copy(
                        bufs[b], out.at[pl.ds(base + j * C, C)],
                        ssems[b])

            def wait_s(b):
                for out in outs:
                    pltpu.make_async_copy(
                        bufs[b], out.at[pl.ds(0, C)], ssems[b]).wait()

            for b in range(G):
                issue(b, b)
            for j in range(G):
                issue(j + G, (j + G) % nbuf)
                wait_g(j % nbuf)
                store(j, j % nbuf)

            steady_n = m - 2 * G
            nk = (steady_n + nbuf - 1) // nbuf

            def body(k, _):
                for u in range(nbuf):
                    j = G + k * nbuf + u

                    @pl.when(j < m - G)
                    def _(j=j, u=u):
                        b = (G + u) % nbuf
                        br = (2 * G + u) % nbuf
                        wait_s(br)
                        issue(j + G, br)
                        wait_g(b)
                        store(j, b)
                return 0

            lax.fori_loop(0, nk, body, 0)
            for j in range(m - G, m):
                wait_g(j % nbuf)
                store(j, j % nbuf)
            for j in range(m - nbuf, m):
                wait_s(j % nbuf)

        pltpu.sync_copy(idx_hbm.at[pl.ds(wid * nidx, nidx)],
                        idx_v.at[pl.ds(0, nidx)])
        pipeline(wid * nidx)

    return gather


def _finish_body(rows_ref, w_ref, b_ref, ey_ref, px_ref, py_ref):
    e = rows_ref[...]
    p = lax.dot_general(
        jnp.maximum(e, 0.0), w_ref[...],
        dimension_numbers=(((1,), (1,)), ((), ())),
        preferred_element_type=jnp.float32,
    ) + b_ref[...]
    ey_ref[...] = e
    px_ref[...] = p
    py_ref[...] = p


def _finish(rows, W, b):
    N = rows.shape[0]
    R = 8192 if N % 8192 == 0 else 4096
    shp = jax.ShapeDtypeStruct((N, D), jnp.float32)
    o2 = pl.BlockSpec((R, D), lambda i: (i, 0))
    return pl.pallas_call(
        _finish_body,
        grid=(N // R,),
        in_specs=[
            pl.BlockSpec((R, D), lambda i: (i, 0)),
            pl.BlockSpec((D, D), lambda i: (0, 0)),
            pl.BlockSpec((1, D), lambda i: (0, 0)),
        ],
        out_specs=[o2, o2, o2],
        out_shape=[shp, shp, shp],
    )(rows, W, b.reshape(1, D))


def kernel(tensor_demo, tensor_med, tensor_vitals, tensor_labs, table, W, b):
    V = table.shape[0]
    tensors = (tensor_demo, tensor_med, tensor_vitals, tensor_labs)
    exs = []
    for t in tensors:
        B, ns = t.shape
        nidx = B * ns // _NW
        C, nbuf = 128, 4
        idx = t.astype(jnp.int32).T.reshape(B * ns)
        exs.append(_make_gather(V, nidx, C, nbuf)(table, idx))

    embs_x, projs_x, embs_y, projs_y = [], [], [], []
    for t, ex in zip(tensors, exs):
        B, ns = t.shape
        ey, px, py = _finish(ex, W, b)
        for acc, a in ((embs_x, ex), (projs_x, px), (embs_y, ey),
                       (projs_y, py)):
            acc.append(jnp.transpose(a.reshape(ns, B, D), (1, 0, 2)))
    return (tuple(embs_x), tuple(projs_x), tuple(embs_y), tuple(projs_y))
